# Initial kernel scaffold; baseline (speedup 1.0000x reference)
import functools

import jax
import jax.numpy as jnp
from jax import lax
from jax.experimental import pallas as pl
from jax.experimental.pallas import tpu as pltpu
from jax.experimental.pallas import tpu_sc as plsc

_N = 10000
_E = 320000
_D = 128
_B = 64
_C = 10

_NC = 2
_NS = 16
_NW = _NC * _NS
_EPT = _E // _NW
_K = 80
_CHUNKS = _EPT // _K
_RPT = _N // _NS


def _sc_scatter_add(h, src3, dst3, zeros):
    mesh = plsc.VectorSubcoreMesh(core_axis_name="c", subcore_axis_name="s")

    @functools.partial(
        pl.kernel,
        out_type=jax.ShapeDtypeStruct((_NC, _N, _D), jnp.float32),
        mesh=mesh,
        scratch_types=[
            pltpu.VMEM((_CHUNKS, _K), jnp.int32),
            pltpu.VMEM((_CHUNKS, _K), jnp.int32),
            pltpu.VMEM((_K, _D), jnp.float32),
            pltpu.VMEM_SHARED((_N, _D), jnp.float32),
            pltpu.SemaphoreType.DMA,
        ],
    )
    def k(h_hbm, src_hbm, dst_hbm, z_hbm, out_hbm, sidx, didx, rows, acc, sem):
        c = lax.axis_index("c")
        s = lax.axis_index("s")
        wid = c * _NS + s
        pltpu.sync_copy(src_hbm.at[wid], sidx)
        pltpu.sync_copy(dst_hbm.at[wid], didx)
        pltpu.sync_copy(z_hbm.at[pl.ds(s * _RPT, _RPT)],
                        acc.at[pl.ds(s * _RPT, _RPT)])
        plsc.subcore_barrier()

        def body(j, carry):
            pltpu.async_copy(h_hbm.at[sidx.at[j]], rows, sem).wait()
            pltpu.sync_copy(rows, acc.at[didx.at[j]], add=True)
            return carry

        lax.fori_loop(0, _CHUNKS, body, 0)
        plsc.subcore_barrier()
        pltpu.sync_copy(acc.at[pl.ds(s * _RPT, _RPT)],
                        out_hbm.at[c, pl.ds(s * _RPT, _RPT)])

    return k(h, src3, dst3, zeros)


def _bn_relu(z, g, be):
    m = jnp.mean(z, axis=0, keepdims=True)
    v = jnp.mean(z * z, axis=0, keepdims=True) - m * m
    return jnp.maximum(g * (z - m) * lax.rsqrt(v + 1e-5) + be, 0.0)


def _gin_mlp(p_ref, h_ref, sc_ref, W1_ref, b1_ref, g1_ref, be1_ref,
             W2_ref, b2_ref, g_ref, be_ref):
    pooled = p_ref[0] + p_ref[1] + sc_ref[0] * h_ref[...]
    z = jnp.dot(pooled, W1_ref[...], preferred_element_type=jnp.float32) + b1_ref[...]
    hid = _bn_relu(z, g1_ref[...], be1_ref[...])
    z2 = jnp.dot(hid, W2_ref[...], preferred_element_type=jnp.float32) + b2_ref[...]
    return _bn_relu(z2, g_ref[...], be_ref[...])


def _layer_body(p_ref, h_ref, sc_ref, W1_ref, b1_ref, g1_ref, be1_ref,
                W2_ref, b2_ref, g_ref, be_ref, o_ref):
    o_ref[...] = _gin_mlp(p_ref, h_ref, sc_ref, W1_ref, b1_ref, g1_ref,
                          be1_ref, W2_ref, b2_ref, g_ref, be_ref)


def _final_body(p_ref, h_ref, sc_ref, W1_ref, b1_ref, g1_ref, be1_ref,
                W2_ref, b2_ref, g_ref, be_ref, gid_ref,
                fc1W_ref, fc1b_ref, fc2W_ref, fc2b_ref, fc3W_ref, fc3b_ref,
                o_ref):
    hfin = _gin_mlp(p_ref, h_ref, sc_ref, W1_ref, b1_ref, g1_ref, be1_ref,
                    W2_ref, b2_ref, g_ref, be_ref)
    ids = gid_ref[...]
    ohT = (lax.broadcasted_iota(jnp.int32, (_B, _N), 0) == ids).astype(jnp.float32)
    gp = jnp.dot(ohT, hfin, preferred_element_type=jnp.float32)
    z = jnp.maximum(jnp.dot(gp, fc1W_ref[...],
                            preferred_element_type=jnp.float32) + fc1b_ref[...], 0.0)
    z = jnp.maximum(jnp.dot(z, fc2W_ref[...],
                            preferred_element_type=jnp.float32) + fc2b_ref[...], 0.0)
    o_ref[...] = jnp.dot(z, fc3W_ref[...],
                         preferred_element_type=jnp.float32) + fc3b_ref[...]


def _tc_layer(p, h, scale, W1, b1, g1, be1, W2, b2, g, be):
    specs = ([pl.BlockSpec(memory_space=pltpu.VMEM)] * 2
             + [pl.BlockSpec(memory_space=pltpu.SMEM)]
             + [pl.BlockSpec(memory_space=pltpu.VMEM)] * 8)
    return pl.pallas_call(
        _layer_body,
        out_shape=jax.ShapeDtypeStruct((_N, _D), jnp.float32),
        in_specs=specs,
        out_specs=pl.BlockSpec(memory_space=pltpu.VMEM),
    )(p, h, scale, W1, b1, g1, be1, W2, b2, g, be)


def _tc_final(p, h, scale, W1, b1, g1, be1, W2, b2, g, be, gids,
              fc1W, fc1b, fc2W, fc2b, fc3W, fc3b):
    specs = ([pl.BlockSpec(memory_space=pltpu.VMEM)] * 2
             + [pl.BlockSpec(memory_space=pltpu.SMEM)]
             + [pl.BlockSpec(memory_space=pltpu.VMEM)] * 15)
    return pl.pallas_call(
        _final_body,
        out_shape=jax.ShapeDtypeStruct((_B, _C), jnp.float32),
        in_specs=specs,
        out_specs=pl.BlockSpec(memory_space=pltpu.VMEM),
    )(p, h, scale, W1, b1, g1, be1, W2, b2, g, be, gids,
      fc1W, fc1b, fc2W, fc2b, fc3W, fc3b)


def kernel(x, edge_index, graph_ids, eps,
           l0_W1, l0_b1, l0_g1, l0_be1, l0_W2, l0_b2, l0_g, l0_be,
           l1_W1, l1_b1, l1_g1, l1_be1, l1_W2, l1_b2, l1_g, l1_be,
           fc1_W, fc1_b, fc2_W, fc2_b, fc3_W, fc3_b):
    src3 = edge_index[0].astype(jnp.int32).reshape(_NW, _CHUNKS, _K)
    dst3 = edge_index[1].astype(jnp.int32).reshape(_NW, _CHUNKS, _K)
    zeros = jnp.zeros((_N, _D), jnp.float32)
    scale0 = (1.0 + eps[0]).reshape(1).astype(jnp.float32)
    scale1 = (1.0 + eps[1]).reshape(1).astype(jnp.float32)
    gids = graph_ids.astype(jnp.int32).reshape(1, _N)

    r = lambda a: a.reshape(1, -1).astype(jnp.float32)

    p0 = _sc_scatter_add(x, src3, dst3, zeros)
    h1 = _tc_layer(p0, x, scale0, l0_W1, r(l0_b1), r(l0_g1), r(l0_be1),
                   l0_W2, r(l0_b2), r(l0_g), r(l0_be))
    p1 = _sc_scatter_add(h1, src3, dst3, zeros)
    return _tc_final(p1, h1, scale1, l1_W1, r(l1_b1), r(l1_g1), r(l1_be1),
                     l1_W2, r(l1_b2), r(l1_g), r(l1_be), gids,
                     fc1_W, r(fc1_b), fc2_W, r(fc2_b), fc3_W, r(fc3_b))


# speedup vs baseline: 6.7861x; 6.7861x over previous
import functools

import jax
import jax.numpy as jnp
from jax import lax
from jax.experimental import pallas as pl
from jax.experimental.pallas import tpu as pltpu
from jax.experimental.pallas import tpu_sc as plsc

_N = 10000
_E = 320000
_D = 128
_B = 64
_C = 10

_NC = 2
_NS = 16
_NW = _NC * _NS
_EPT = _E // _NW
_K = 80
_CHUNKS = _EPT // _K
_RPT = 632
_NP = _RPT * _NS


def _sc_scatter_add(h, src3, dst3, zeros):
    mesh = plsc.VectorSubcoreMesh(core_axis_name="c", subcore_axis_name="s")

    @functools.partial(
        pl.kernel,
        out_type=jax.ShapeDtypeStruct((_NC, _NP, _D), jnp.float32),
        mesh=mesh,
        scratch_types=[
            pltpu.VMEM((_CHUNKS, _K), jnp.int32),
            pltpu.VMEM((_CHUNKS, _K), jnp.int32),
            pltpu.VMEM((_K, _D), jnp.float32),
            pltpu.VMEM_SHARED((_NP, _D), jnp.float32),
            pltpu.SemaphoreType.DMA,
        ],
    )
    def k(h_hbm, src_hbm, dst_hbm, z_hbm, out_hbm, sidx, didx, rows, acc, sem):
        c = lax.axis_index("c")
        s = lax.axis_index("s")
        wid = c * _NS + s
        pltpu.sync_copy(src_hbm.at[wid], sidx)
        pltpu.sync_copy(dst_hbm.at[wid], didx)
        pltpu.sync_copy(z_hbm.at[pl.ds(s * _RPT, _RPT)],
                        acc.at[pl.ds(s * _RPT, _RPT)])
        plsc.subcore_barrier()

        def body(j, carry):
            pltpu.async_copy(h_hbm.at[sidx.at[j]], rows, sem).wait()
            pltpu.sync_copy(rows, acc.at[didx.at[j]], add=True)
            return carry

        lax.fori_loop(0, _CHUNKS, body, 0)
        plsc.subcore_barrier()
        pltpu.sync_copy(acc.at[pl.ds(s * _RPT, _RPT)],
                        out_hbm.at[c, pl.ds(s * _RPT, _RPT)])

    return k(h, src3, dst3, zeros)


def _bn_relu(z, g, be):
    m = jnp.mean(z, axis=0, keepdims=True)
    v = jnp.mean(z * z, axis=0, keepdims=True) - m * m
    return jnp.maximum(g * (z - m) * lax.rsqrt(v + 1e-5) + be, 0.0)


def _gin_mlp(p_ref, h_ref, sc_ref, W1_ref, b1_ref, g1_ref, be1_ref,
             W2_ref, b2_ref, g_ref, be_ref):
    pooled = p_ref[0, :_N] + p_ref[1, :_N] + sc_ref[0] * h_ref[...]
    z = jnp.dot(pooled, W1_ref[...], preferred_element_type=jnp.float32) + b1_ref[...]
    hid = _bn_relu(z, g1_ref[...], be1_ref[...])
    z2 = jnp.dot(hid, W2_ref[...], preferred_element_type=jnp.float32) + b2_ref[...]
    return _bn_relu(z2, g_ref[...], be_ref[...])


def _layer_body(p_ref, h_ref, sc_ref, W1_ref, b1_ref, g1_ref, be1_ref,
                W2_ref, b2_ref, g_ref, be_ref, o_ref):
    o_ref[...] = _gin_mlp(p_ref, h_ref, sc_ref, W1_ref, b1_ref, g1_ref,
                          be1_ref, W2_ref, b2_ref, g_ref, be_ref)


def _final_body(p_ref, h_ref, sc_ref, W1_ref, b1_ref, g1_ref, be1_ref,
                W2_ref, b2_ref, g_ref, be_ref, gid_ref,
                fc1W_ref, fc1b_ref, fc2W_ref, fc2b_ref, fc3W_ref, fc3b_ref,
                o_ref):
    hfin = _gin_mlp(p_ref, h_ref, sc_ref, W1_ref, b1_ref, g1_ref, be1_ref,
                    W2_ref, b2_ref, g_ref, be_ref)
    ids = gid_ref[...]
    ohT = (lax.broadcasted_iota(jnp.int32, (_B, _N), 0) == ids).astype(jnp.float32)
    gp = jnp.dot(ohT, hfin, preferred_element_type=jnp.float32)
    z = jnp.maximum(jnp.dot(gp, fc1W_ref[...],
                            preferred_element_type=jnp.float32) + fc1b_ref[...], 0.0)
    z = jnp.maximum(jnp.dot(z, fc2W_ref[...],
                            preferred_element_type=jnp.float32) + fc2b_ref[...], 0.0)
    o_ref[...] = jnp.dot(z, fc3W_ref[...],
                         preferred_element_type=jnp.float32) + fc3b_ref[...]


def _tc_layer(p, h, scale, W1, b1, g1, be1, W2, b2, g, be):
    specs = ([pl.BlockSpec(memory_space=pltpu.VMEM)] * 2
             + [pl.BlockSpec(memory_space=pltpu.SMEM)]
             + [pl.BlockSpec(memory_space=pltpu.VMEM)] * 8)
    return pl.pallas_call(
        _layer_body,
        out_shape=jax.ShapeDtypeStruct((_N, _D), jnp.float32),
        in_specs=specs,
        out_specs=pl.BlockSpec(memory_space=pltpu.VMEM),
    )(p, h, scale, W1, b1, g1, be1, W2, b2, g, be)


def _tc_final(p, h, scale, W1, b1, g1, be1, W2, b2, g, be, gids,
              fc1W, fc1b, fc2W, fc2b, fc3W, fc3b):
    specs = ([pl.BlockSpec(memory_space=pltpu.VMEM)] * 2
             + [pl.BlockSpec(memory_space=pltpu.SMEM)]
             + [pl.BlockSpec(memory_space=pltpu.VMEM)] * 15)
    return pl.pallas_call(
        _final_body,
        out_shape=jax.ShapeDtypeStruct((_B, _C), jnp.float32),
        in_specs=specs,
        out_specs=pl.BlockSpec(memory_space=pltpu.VMEM),
    )(p, h, scale, W1, b1, g1, be1, W2, b2, g, be, gids,
      fc1W, fc1b, fc2W, fc2b, fc3W, fc3b)


def kernel(x, edge_index, graph_ids, eps,
           l0_W1, l0_b1, l0_g1, l0_be1, l0_W2, l0_b2, l0_g, l0_be,
           l1_W1, l1_b1, l1_g1, l1_be1, l1_W2, l1_b2, l1_g, l1_be,
           fc1_W, fc1_b, fc2_W, fc2_b, fc3_W, fc3_b):
    src3 = edge_index[0].astype(jnp.int32).reshape(_NW, _CHUNKS, _K)
    dst3 = edge_index[1].astype(jnp.int32).reshape(_NW, _CHUNKS, _K)
    zeros = jnp.zeros((_NP, _D), jnp.float32)
    scale0 = (1.0 + eps[0]).reshape(1).astype(jnp.float32)
    scale1 = (1.0 + eps[1]).reshape(1).astype(jnp.float32)
    gids = graph_ids.astype(jnp.int32).reshape(1, _N)

    r = lambda a: a.reshape(1, -1).astype(jnp.float32)

    p0 = _sc_scatter_add(x, src3, dst3, zeros)
    h1 = _tc_layer(p0, x, scale0, l0_W1, r(l0_b1), r(l0_g1), r(l0_be1),
                   l0_W2, r(l0_b2), r(l0_g), r(l0_be))
    p1 = _sc_scatter_add(h1, src3, dst3, zeros)
    return _tc_final(p1, h1, scale1, l1_W1, r(l1_b1), r(l1_g1), r(l1_be1),
                     l1_W2, r(l1_b2), r(l1_g), r(l1_be), gids,
                     fc1_W, r(fc1_b), fc2_W, r(fc2_b), fc3_W, r(fc3_b))
